# baseline (speedup 1.0000x reference)
import functools

import jax
import jax.numpy as jnp
from jax import lax
from jax.experimental import pallas as pl
from jax.experimental.pallas import tpu as pltpu
from jax.experimental.pallas import tpu_sc as plsc

VOCAB = 1000000
DIM = 64
B = 16384

CHUNK = 128


def _build():
    info = plsc.get_sparse_core_info()
    nc, ns = info.num_cores, info.num_subcores
    nw = nc * ns
    b_per_w = B // nw
    n_chunks = b_per_w // CHUNK
    mesh = plsc.VectorSubcoreMesh(core_axis_name="c", subcore_axis_name="s")

    out_t = jax.ShapeDtypeStruct((B, DIM), jnp.float32)

    @functools.partial(
        pl.kernel,
        mesh=mesh,
        out_type=[out_t, out_t, out_t],
        scratch_types=(
            [pltpu.VMEM((n_chunks, CHUNK), jnp.int32)] * 3
            + [pltpu.VMEM((b_per_w, DIM), jnp.float32)] * 3
            + [pltpu.SemaphoreType.DMA] * 3
        ),
        compiler_params=pltpu.CompilerParams(use_tc_tiling_on_sc=False),
    )
    def triple_gather(a_hbm, p_hbm, n_hbm, table_hbm,
                      oa_hbm, op_hbm, on_hbm,
                      ia, ip, in_, ra, rp, rn,
                      sem_idx, sem_g, sem_out):
        wid = lax.axis_index("s") * nc + lax.axis_index("c")
        base = wid * b_per_w

        idx_refs = (ia, ip, in_)
        row_refs = (ra, rp, rn)
        src_refs = (a_hbm, p_hbm, n_hbm)
        out_refs = (oa_hbm, op_hbm, on_hbm)

        idx_cps = []
        for s in range(3):
            for j in range(n_chunks):
                idx_cps.append(pltpu.async_copy(
                    src_refs[s].at[pl.ds(base + j * CHUNK, CHUNK)],
                    idx_refs[s].at[j], sem_idx))
        for cp in idx_cps:
            cp.wait()

        g_cps = []
        for s in range(3):
            for j in range(n_chunks):
                g_cps.append(pltpu.async_copy(
                    table_hbm.at[idx_refs[s].at[j]],
                    row_refs[s].at[pl.ds(j * CHUNK, CHUNK)], sem_g))
        for cp in g_cps:
            cp.wait()

        o_cps = []
        for s in range(3):
            o_cps.append(pltpu.async_copy(
                row_refs[s], out_refs[s].at[pl.ds(base, b_per_w)], sem_out))
        for cp in o_cps:
            cp.wait()

    return triple_gather


_TRIPLE_GATHER = _build()


@jax.jit
def kernel(anchor, pos, neg, table):
    a = anchor.astype(jnp.int32)
    p = pos.astype(jnp.int32)
    n = neg.astype(jnp.int32)
    oa, op_, on = _TRIPLE_GATHER(a, p, n, table)
    return (oa.reshape(-1, 1), op_.reshape(-1, 1), on.reshape(-1, 1))


# speedup vs baseline: 1.8919x; 1.8919x over previous
import functools

import jax
import jax.numpy as jnp
from jax import lax
from jax.experimental import pallas as pl
from jax.experimental.pallas import tpu as pltpu
from jax.experimental.pallas import tpu_sc as plsc

VOCAB = 1000000
DIM = 64
B = 16384
CHUNK = 32


def _build():
    info = plsc.get_sparse_core_info()
    nc, ns = info.num_cores, info.num_subcores
    nw = nc * ns
    b_per_w = B // nw
    n_chunks = b_per_w // CHUNK
    mesh = plsc.VectorSubcoreMesh(core_axis_name="c", subcore_axis_name="s")

    out_t = jax.ShapeDtypeStruct((B * DIM,), jnp.float32)

    @functools.partial(
        pl.kernel,
        mesh=mesh,
        out_type=[out_t, out_t, out_t],
        scratch_types=(
            [pltpu.VMEM((b_per_w + 16,), jnp.int32)]
            + [pltpu.VMEM((CHUNK, 8, DIM), jnp.float32)] * 2
            + [pltpu.VMEM((CHUNK * DIM,), jnp.float32)] * 2
            + [pltpu.SemaphoreType.DMA] * 3
        ),
        compiler_params=pltpu.CompilerParams(use_tc_tiling_on_sc=True),
    )
    def triple_gather(a_hbm, p_hbm, n_hbm, table3_hbm,
                      oa_hbm, op_hbm, on_hbm,
                      idx_v, slab0, slab1, rows0, rows1,
                      sem0, sem1, sem_out):
        wid = lax.axis_index("s") * nc + lax.axis_index("c")
        base = wid * b_per_w

        src_refs = (a_hbm, p_hbm, n_hbm)
        out_refs = (oa_hbm, op_hbm, on_hbm)
        slabs = (slab0, slab1)
        sems = (sem0, sem1)

        for s in range(3):
            pltpu.sync_copy(src_refs[s].at[pl.ds(base, b_per_w)],
                            idx_v.at[pl.ds(0, b_per_w)])

            def fire(j, buf):
                def body(k, _, j=j, buf=buf):
                    v = idx_v[pl.ds(j * CHUNK + k, 16)]
                    t = lax.div(v[0], 8)
                    pltpu.async_copy(table3_hbm.at[t], slabs[buf].at[k],
                                     sems[buf])
                    return ()

                lax.fori_loop(0, CHUNK, body, (), unroll=False)

            def drain(buf):
                pltpu.make_async_copy(table3_hbm.at[pl.ds(0, CHUNK)],
                                      slabs[buf], sems[buf]).wait()

            rows = (rows0, rows1)
            out_cps = [None, None]
            fire(0, 0)
            for j in range(n_chunks):
                if j + 1 < n_chunks:
                    fire(j + 1, (j + 1) % 2)
                drain(j % 2)
                slab = slabs[j % 2]
                rbuf = rows[j % 2]
                if out_cps[j % 2] is not None:
                    out_cps[j % 2].wait()

                def extract(k, _, j=j, slab=slab, rbuf=rbuf):
                    v = idx_v[pl.ds(j * CHUNK + k, 16)]
                    sub = lax.rem(v[0], 8)
                    for c in range(DIM // 16):
                        rbuf[pl.ds(k * DIM + c * 16, 16)] = (
                            slab[k, sub, pl.ds(c * 16, 16)])
                    return ()

                lax.fori_loop(0, CHUNK, extract, (), unroll=False)
                out_cps[j % 2] = pltpu.async_copy(
                    rbuf,
                    out_refs[s].at[pl.ds((base + j * CHUNK) * DIM,
                                         CHUNK * DIM)],
                    sem_out)
            for cp in out_cps:
                if cp is not None:
                    cp.wait()

    return triple_gather


_TRIPLE_GATHER = _build()


@jax.jit
def kernel(anchor, pos, neg, table):
    a = anchor.astype(jnp.int32)
    p = pos.astype(jnp.int32)
    n = neg.astype(jnp.int32)
    t3 = table.reshape(VOCAB // 8, 8, DIM)
    oa, op_, on = _TRIPLE_GATHER(a, p, n, t3)
    return (oa.reshape(-1, 1), op_.reshape(-1, 1), on.reshape(-1, 1))
